# Initial kernel scaffold; baseline (speedup 1.0000x reference)
import jax
import jax.numpy as jnp
from jax.experimental import pallas as pl


def kernel(sequence_output, attention, entity_pos, hts):
    raise NotImplementedError("write your pallas kernel here")


# speedup vs baseline: 8.4463x; 8.4463x over previous
import jax
import jax.numpy as jnp
from jax.experimental import pallas as pl


_N, _L, _D, _H, _E, _M, _P = 4, 512, 768, 12, 64, 3, 1024


def _hrt_kernel(pos_ref, hidx_ref, tidx_ref, seq_ref, att_ref,
                hs_ref, ts_ref, rs_ref):
    seq = seq_ref[0]
    pos = pos_ref[0, 0, :]
    hidx = hidx_ref[0, 0, :]
    tidx = tidx_ref[0, 0, :]

    l_iota = jax.lax.broadcasted_iota(jnp.int32, (_E * _M, _L), 1)
    poh = (pos[:, None] == l_iota).astype(jnp.float32)

    mention = jnp.dot(poh, seq, preferred_element_type=jnp.float32)
    me = mention.reshape(_E, _M, _D)
    mmax = jnp.max(me, axis=1)
    e_emb = mmax + jnp.log(jnp.sum(jnp.exp(me - mmax[:, None, :]), axis=1))

    w = poh.reshape(_E, _M, _L).sum(axis=1) * (1.0 / _M)

    e_iota = jax.lax.broadcasted_iota(jnp.int32, (_P, _E), 1)
    oh_h = (hidx[:, None] == e_iota).astype(jnp.float32)
    oh_t = (tidx[:, None] == e_iota).astype(jnp.float32)

    acc = jnp.zeros((_P, _L), jnp.float32)
    for hh in range(_H):
        att_h = att_ref[0, hh]
        e_att_h = jnp.dot(w, att_h, preferred_element_type=jnp.float32)
        h_att = jnp.dot(oh_h, e_att_h, preferred_element_type=jnp.float32)
        t_att = jnp.dot(oh_t, e_att_h, preferred_element_type=jnp.float32)
        acc = acc + h_att * t_att

    ht_att = acc * (1.0 / _H)
    ht_att = ht_att / (jnp.sum(ht_att, axis=1, keepdims=True) + 1e-5)

    rs_ref[0] = jnp.dot(ht_att, seq, preferred_element_type=jnp.float32)
    hs_ref[0] = jnp.dot(oh_h, e_emb, preferred_element_type=jnp.float32)
    ts_ref[0] = jnp.dot(oh_t, e_emb, preferred_element_type=jnp.float32)


def kernel(sequence_output, attention, entity_pos, hts):
    n, L, d = sequence_output.shape
    h = attention.shape[1]
    E, M = entity_pos.shape[1], entity_pos.shape[2]
    P = hts.shape[1]
    assert (n, L, d, h, E, M, P) == (_N, _L, _D, _H, _E, _M, _P)

    pos = (entity_pos[:, :, :, 0].reshape(n, 1, E * M) + 1).astype(jnp.int32)
    hidx = hts[:, :, 0].reshape(n, 1, P).astype(jnp.int32)
    tidx = hts[:, :, 1].reshape(n, 1, P).astype(jnp.int32)

    out_shape = [jax.ShapeDtypeStruct((n, P, d), jnp.float32)] * 3
    hs, ts, rs = pl.pallas_call(
        _hrt_kernel,
        grid=(n,),
        in_specs=[
            pl.BlockSpec((1, 1, E * M), lambda i: (i, 0, 0)),
            pl.BlockSpec((1, 1, P), lambda i: (i, 0, 0)),
            pl.BlockSpec((1, 1, P), lambda i: (i, 0, 0)),
            pl.BlockSpec((1, L, d), lambda i: (i, 0, 0)),
            pl.BlockSpec((1, h, L, L), lambda i: (i, 0, 0, 0)),
        ],
        out_specs=[
            pl.BlockSpec((1, P, d), lambda i: (i, 0, 0)),
            pl.BlockSpec((1, P, d), lambda i: (i, 0, 0)),
            pl.BlockSpec((1, P, d), lambda i: (i, 0, 0)),
        ],
        out_shape=out_shape,
    )(pos, hidx, tidx, sequence_output, attention)

    return hs.reshape(-1, d), ts.reshape(-1, d), rs.reshape(-1, d)
